# baseline (speedup 1.0000x reference)
import functools

import jax
import jax.numpy as jnp
from jax import lax
from jax.experimental import pallas as pl
from jax.experimental.pallas import tpu as pltpu
from jax.experimental.pallas import tpu_sc as plsc

EMBED_DIM = 64
PACK_W = 2 * EMBED_DIM
NBUF = 2
PACK_COLS = 2048


@functools.lru_cache(maxsize=None)
def _make_pack(vocab: int):
    grid = pl.cdiv(vocab, PACK_COLS)

    @functools.partial(
        pl.pallas_call,
        grid=(grid,),
        in_specs=[pl.BlockSpec((EMBED_DIM, PACK_COLS), lambda i: (0, i))],
        out_specs=pl.BlockSpec((PACK_COLS, PACK_W), lambda i: (i, 0)),
        out_shape=jax.ShapeDtypeStruct((vocab, PACK_W), jnp.float32),
    )
    def pack_kernel(tin, tout):
        x = tin[...]
        r = lax.broadcasted_iota(jnp.int32, (EMBED_DIM, EMBED_DIM), 0)
        c = lax.broadcasted_iota(jnp.int32, (EMBED_DIM, EMBED_DIM), 1)
        eye = (r == c).astype(jnp.float32)
        y = lax.dot_general(x, eye, (((0,), (0,)), ((), ())),
                            preferred_element_type=jnp.float32)
        tout[:, 0:EMBED_DIM] = y

    return pack_kernel


@functools.lru_cache(maxsize=None)
def _make_gather(bsz: int, seq: int, vocab: int):
    n_tokens = bsz * seq
    info = plsc.get_sparse_core_info()
    nc, ns = info.num_cores, info.num_subcores
    nw = nc * ns
    b_per_w = n_tokens // nw
    chunk = seq
    n_chunks = b_per_w // chunk
    assert n_tokens % (nw * chunk) == 0
    mesh = plsc.VectorSubcoreMesh(core_axis_name="c", subcore_axis_name="s")

    @functools.partial(
        pl.kernel,
        mesh=mesh,
        out_type=jax.ShapeDtypeStruct((bsz, seq, EMBED_DIM), jnp.float32),
        scratch_types=[
            pltpu.VMEM((b_per_w,), jnp.int32),
            pltpu.VMEM((NBUF, chunk, PACK_W), jnp.float32),
            pltpu.VMEM((NBUF, chunk, EMBED_DIM), jnp.float32),
            pltpu.SemaphoreType.DMA((NBUF,)),
            pltpu.SemaphoreType.DMA((NBUF,)),
        ],
        compiler_params=pltpu.CompilerParams(
            needs_layout_passes=False,
            disable_bounds_checks=True,
        ),
    )
    def gather_kernel(idx_hbm, tbl_hbm, out_hbm,
                      idx_v, rows_v, stage_v, gsem, ssem):
        wid = lax.axis_index("s") * nc + lax.axis_index("c")
        base = wid * b_per_w
        pltpu.sync_copy(idx_hbm.at[pl.ds(base, b_per_w)], idx_v)

        def issue_gather(j):
            b = j % NBUF
            return pltpu.async_copy(
                tbl_hbm.at[idx_v.at[pl.ds(j * chunk, chunk)]],
                rows_v.at[b],
                gsem.at[b],
            )

        def issue_store(j):
            b = j % NBUF
            return pltpu.async_copy(
                stage_v.at[b],
                out_hbm.at[wid * n_chunks + j],
                ssem.at[b],
            )

        gd = [None] * n_chunks
        sd = [None] * n_chunks
        for j in range(min(2, n_chunks)):
            gd[j] = issue_gather(j)
        for j in range(n_chunks):
            b = j % NBUF
            gd[j].wait()
            if j >= NBUF:
                sd[j - NBUF].wait()
            def tok_body(t, carry):
                for q in range(EMBED_DIM // 16):
                    stage_v[b, t, pl.ds(q * 16, 16)] = (
                        rows_v[b, t, pl.ds(q * 16, 16)])
                return carry

            lax.fori_loop(0, chunk, tok_body, 0)
            sd[j] = issue_store(j)
            if j + 2 < n_chunks:
                gd[j + 2] = issue_gather(j + 2)
        for j in range(max(0, n_chunks - NBUF), n_chunks):
            sd[j].wait()

    return gather_kernel


def kernel(token_ids, table):
    b, s = token_ids.shape
    vocab, d = table.shape
    assert d == EMBED_DIM
    flat = token_ids.reshape(b * s)
    packed = _make_pack(vocab)(jnp.swapaxes(table, 0, 1))
    return _make_gather(b, s, vocab)(flat, packed)


# speedup vs baseline: 1.5115x; 1.5115x over previous
import functools

import jax
import jax.numpy as jnp
from jax import lax
from jax.experimental import pallas as pl
from jax.experimental.pallas import tpu as pltpu
from jax.experimental.pallas import tpu_sc as plsc

EMBED_DIM = 64
PACK_W = 2 * EMBED_DIM
NBUF = 2
PACK_COLS = 8192


@functools.lru_cache(maxsize=None)
def _make_pack(vocab: int):
    grid = pl.cdiv(vocab, PACK_COLS)

    @functools.partial(
        pl.pallas_call,
        grid=(grid,),
        in_specs=[pl.BlockSpec((EMBED_DIM, PACK_COLS), lambda i: (0, i))],
        out_specs=pl.BlockSpec((PACK_COLS, PACK_W), lambda i: (i, 0)),
        out_shape=jax.ShapeDtypeStruct((vocab, PACK_W), jnp.float32),
    )
    def pack_kernel(tin, tout):
        tout[:, 0:EMBED_DIM] = jnp.transpose(tin[...])

    return pack_kernel


@functools.lru_cache(maxsize=None)
def _make_gather(bsz: int, seq: int, vocab: int):
    n_tokens = bsz * seq
    info = plsc.get_sparse_core_info()
    nc, ns = info.num_cores, info.num_subcores
    nw = nc * ns
    b_per_w = n_tokens // nw
    chunk = seq
    n_chunks = b_per_w // chunk
    assert n_tokens % (nw * chunk) == 0
    mesh = plsc.VectorSubcoreMesh(core_axis_name="c", subcore_axis_name="s")

    @functools.partial(
        pl.kernel,
        mesh=mesh,
        out_type=jax.ShapeDtypeStruct((bsz, seq, EMBED_DIM), jnp.float32),
        scratch_types=[
            pltpu.VMEM((b_per_w,), jnp.int32),
            pltpu.VMEM((NBUF, chunk, PACK_W), jnp.float32),
            pltpu.VMEM((NBUF, chunk, EMBED_DIM), jnp.float32),
            pltpu.SemaphoreType.DMA((NBUF,)),
            pltpu.SemaphoreType.DMA((NBUF,)),
        ],
        compiler_params=pltpu.CompilerParams(
            needs_layout_passes=False,
            disable_bounds_checks=True,
        ),
    )
    def gather_kernel(idx_hbm, tbl_hbm, out_hbm,
                      idx_v, rows_v, stage_v, gsem, ssem):
        wid = lax.axis_index("s") * nc + lax.axis_index("c")
        base = wid * b_per_w
        pltpu.sync_copy(idx_hbm.at[pl.ds(base, b_per_w)], idx_v)

        def issue_gather(j):
            b = j % NBUF
            return pltpu.async_copy(
                tbl_hbm.at[idx_v.at[pl.ds(j * chunk, chunk)]],
                rows_v.at[b],
                gsem.at[b],
            )

        def issue_store(j):
            b = j % NBUF
            return pltpu.async_copy(
                stage_v.at[b],
                out_hbm.at[wid * n_chunks + j],
                ssem.at[b],
            )

        gd = [None] * n_chunks
        sd = [None] * n_chunks
        for j in range(min(2, n_chunks)):
            gd[j] = issue_gather(j)
        for j in range(n_chunks):
            b = j % NBUF
            gd[j].wait()
            if j >= NBUF:
                sd[j - NBUF].wait()
            def tok_body(t, carry):
                for q in range(EMBED_DIM // 16):
                    stage_v[b, t, pl.ds(q * 16, 16)] = (
                        rows_v[b, t, pl.ds(q * 16, 16)])
                return carry

            lax.fori_loop(0, chunk, tok_body, 0)
            sd[j] = issue_store(j)
            if j + 2 < n_chunks:
                gd[j + 2] = issue_gather(j + 2)
        for j in range(max(0, n_chunks - NBUF), n_chunks):
            sd[j].wait()

    return gather_kernel


def kernel(token_ids, table):
    b, s = token_ids.shape
    vocab, d = table.shape
    assert d == EMBED_DIM
    flat = token_ids.reshape(b * s)
    packed = _make_pack(vocab)(jnp.swapaxes(table, 0, 1))
    return _make_gather(b, s, vocab)(flat, packed)


# speedup vs baseline: 1.6128x; 1.0670x over previous
import functools

import jax
import jax.numpy as jnp
from jax import lax
from jax.experimental import pallas as pl
from jax.experimental.pallas import tpu as pltpu
from jax.experimental.pallas import tpu_sc as plsc

EMBED_DIM = 64
PACK_W = 2 * EMBED_DIM
NBUF = 2
PACK_COLS = 32768


@functools.lru_cache(maxsize=None)
def _make_pack(vocab: int):
    grid = pl.cdiv(vocab, PACK_COLS)

    @functools.partial(
        pl.pallas_call,
        grid=(grid,),
        in_specs=[pl.BlockSpec((EMBED_DIM, PACK_COLS), lambda i: (0, i))],
        out_specs=pl.BlockSpec((PACK_COLS, PACK_W), lambda i: (i, 0)),
        out_shape=jax.ShapeDtypeStruct((vocab, PACK_W), jnp.float32),
    )
    def pack_kernel(tin, tout):
        tout[:, 0:EMBED_DIM] = jnp.transpose(tin[...])

    return pack_kernel


@functools.lru_cache(maxsize=None)
def _make_gather(bsz: int, seq: int, vocab: int):
    n_tokens = bsz * seq
    info = plsc.get_sparse_core_info()
    nc, ns = info.num_cores, info.num_subcores
    nw = nc * ns
    b_per_w = n_tokens // nw
    chunk = seq
    n_chunks = b_per_w // chunk
    assert n_tokens % (nw * chunk) == 0
    mesh = plsc.VectorSubcoreMesh(core_axis_name="c", subcore_axis_name="s")

    @functools.partial(
        pl.kernel,
        mesh=mesh,
        out_type=jax.ShapeDtypeStruct((bsz, seq, EMBED_DIM), jnp.float32),
        scratch_types=[
            pltpu.VMEM((b_per_w,), jnp.int32),
            pltpu.VMEM((NBUF, chunk, PACK_W), jnp.float32),
            pltpu.VMEM((NBUF, chunk, EMBED_DIM), jnp.float32),
            pltpu.SemaphoreType.DMA((NBUF,)),
            pltpu.SemaphoreType.DMA((NBUF,)),
        ],
        compiler_params=pltpu.CompilerParams(
            needs_layout_passes=False,
            disable_bounds_checks=True,
        ),
    )
    def gather_kernel(idx_hbm, tbl_hbm, out_hbm,
                      idx_v, rows_v, stage_v, gsem, ssem):
        wid = lax.axis_index("s") * nc + lax.axis_index("c")
        base = wid * b_per_w
        pltpu.sync_copy(idx_hbm.at[pl.ds(base, b_per_w)], idx_v)

        def issue_gather(j):
            b = j % NBUF
            return pltpu.async_copy(
                tbl_hbm.at[idx_v.at[pl.ds(j * chunk, chunk)]],
                rows_v.at[b],
                gsem.at[b],
            )

        def issue_store(j):
            b = j % NBUF
            return pltpu.async_copy(
                stage_v.at[b],
                out_hbm.at[wid * n_chunks + j],
                ssem.at[b],
            )

        gd = [None] * n_chunks
        sd = [None] * n_chunks
        for j in range(min(2, n_chunks)):
            gd[j] = issue_gather(j)
        for j in range(n_chunks):
            b = j % NBUF
            gd[j].wait()
            if j >= NBUF:
                sd[j - NBUF].wait()
            def tok_body(t, carry):
                for q in range(EMBED_DIM // 16):
                    stage_v[b, t, pl.ds(q * 16, 16)] = (
                        rows_v[b, t, pl.ds(q * 16, 16)])
                return carry

            lax.fori_loop(0, chunk, tok_body, 0)
            sd[j] = issue_store(j)
            if j + 2 < n_chunks:
                gd[j + 2] = issue_gather(j + 2)
        for j in range(max(0, n_chunks - NBUF), n_chunks):
            sd[j].wait()

    return gather_kernel


def kernel(token_ids, table):
    b, s = token_ids.shape
    vocab, d = table.shape
    assert d == EMBED_DIM
    flat = token_ids.reshape(b * s)
    packed = _make_pack(vocab)(jnp.swapaxes(table, 0, 1))
    return _make_gather(b, s, vocab)(flat, packed)
